# baseline (speedup 1.0000x reference)
import functools

import jax
import jax.numpy as jnp
from jax import lax
from jax.experimental import pallas as pl
from jax.experimental.pallas import tpu as pltpu
from jax.experimental.pallas import tpu_sc as plsc

_SC_CORES = 2
_SC_SUBCORES = 16
_NW = _SC_CORES * _SC_SUBCORES


def _routing(species, N, E, BM, G):
    R = G * BM
    one_hot = (species[:, None] == jnp.arange(E, dtype=species.dtype)).astype(
        jnp.int32
    )
    counts = one_hot.sum(axis=0)
    within = jnp.take_along_axis(
        jnp.cumsum(one_hot, axis=0) - one_hot, species[:, None].astype(jnp.int32), axis=1
    )[:, 0]
    nblk = (counts + BM - 1) // BM
    blk_end = jnp.cumsum(nblk)
    padded_start = (blk_end - nblk) * BM
    pos = (padded_start[species] + within).astype(jnp.int32)
    g = jnp.arange(G, dtype=jnp.int32)
    e_of_g = jnp.minimum(
        (g[:, None] >= blk_end[None, :]).sum(axis=1), E - 1
    ).astype(jnp.int32)
    nvalid = blk_end[E - 1]
    next_e = jnp.where(
        blk_end[e_of_g] >= nvalid,
        e_of_g,
        e_of_g[jnp.clip(blk_end[e_of_g], 0, G - 1)],
    ).astype(jnp.int32)
    trans = jnp.concatenate(
        [jnp.zeros((1,), jnp.int32), (e_of_g[1:] != e_of_g[:-1]).astype(jnp.int32)]
    )
    parity = (jnp.cumsum(trans) % 2).astype(jnp.int32)
    nvalid_arr = jnp.full((G,), nvalid, dtype=jnp.int32)
    return e_of_g, next_e, parity, nvalid_arr, pos


def _chunk_rows(b_per_w, row_bytes, budget=230 * 1024):
    cmax = max(8, min(b_per_w, budget // row_bytes))
    for c in range(cmax - cmax % 8, 0, -8):
        if b_per_w % c == 0:
            return c
    return 8


def _sc_disperse(table, idx3, R):
    NT, D = table.shape
    _, NCH, C = idx3.shape
    b_per_w = NT // _NW
    mesh = plsc.VectorSubcoreMesh(core_axis_name="c", subcore_axis_name="s")

    @functools.partial(
        pl.kernel,
        mesh=mesh,
        out_type=jax.ShapeDtypeStruct((R, D), table.dtype),
        scratch_types=[
            pltpu.VMEM((NCH, C), jnp.int32),
            pltpu.VMEM((C, D), table.dtype),
            pltpu.VMEM((C, D), table.dtype),
            pltpu.SemaphoreType.DMA,
            pltpu.SemaphoreType.DMA,
        ],
    )
    def disperse_k(table_hbm, idx_hbm, out_hbm, idx_v, r0, r1, rsem, wsem):
        wid = lax.axis_index("s") * _SC_CORES + lax.axis_index("c")
        base = wid * b_per_w
        rows_v = (r0, r1)
        reads = [None] * NCH
        writes = [None] * NCH
        pltpu.sync_copy(idx_hbm.at[wid], idx_v)
        reads[0] = pltpu.async_copy(table_hbm.at[pl.ds(base, C)], rows_v[0], rsem)
        for j in range(NCH):
            if j + 1 < NCH:
                if j >= 1:
                    writes[j - 1].wait()
                reads[j + 1] = pltpu.async_copy(
                    table_hbm.at[pl.ds(base + (j + 1) * C, C)],
                    rows_v[(j + 1) % 2],
                    rsem,
                )
            reads[j].wait()
            writes[j] = pltpu.async_copy(
                rows_v[j % 2], out_hbm.at[idx_v.at[j]], wsem
            )
        for j in range(max(0, NCH - 2), NCH):
            writes[j].wait()

    return disperse_k(table, idx3)


def _sc_gather(table, idx3):
    _, D = table.shape
    NW, NCH, C = idx3.shape
    b_per_w = NCH * C
    NT = NW * b_per_w
    mesh = plsc.VectorSubcoreMesh(core_axis_name="c", subcore_axis_name="s")

    @functools.partial(
        pl.kernel,
        mesh=mesh,
        out_type=jax.ShapeDtypeStruct((NT, D), table.dtype),
        scratch_types=[
            pltpu.VMEM((NCH, C), jnp.int32),
            pltpu.VMEM((C, D), table.dtype),
            pltpu.VMEM((C, D), table.dtype),
            pltpu.SemaphoreType.DMA,
            pltpu.SemaphoreType.DMA,
        ],
    )
    def gather_k(table_hbm, idx_hbm, out_hbm, idx_v, r0, r1, gsem, wsem):
        wid = lax.axis_index("s") * _SC_CORES + lax.axis_index("c")
        base = wid * b_per_w
        rows_v = (r0, r1)
        gathers = [None] * NCH
        writes = [None] * NCH
        pltpu.sync_copy(idx_hbm.at[wid], idx_v)
        gathers[0] = pltpu.async_copy(table_hbm.at[idx_v.at[0]], rows_v[0], gsem)
        for j in range(NCH):
            if j + 1 < NCH:
                if j >= 1:
                    writes[j - 1].wait()
                gathers[j + 1] = pltpu.async_copy(
                    table_hbm.at[idx_v.at[j + 1]], rows_v[(j + 1) % 2], gsem
                )
            gathers[j].wait()
            writes[j] = pltpu.async_copy(
                rows_v[j % 2], out_hbm.at[pl.ds(base + j * C, C)], wsem
            )
        for j in range(max(0, NCH - 2), NCH):
            writes[j].wait()

    return gather_k(table, idx3)


def _mlp_body(
    e_ref, ne_ref, par_ref, nv_ref, x_ref, w1_ref, b1_ref, w2_ref, b2_ref, y_ref,
    w1f, w2f, w1c, w2c, sem1, sem2,
):
    g = pl.program_id(0)
    e = e_ref[g]
    prev = jnp.where(g > 0, e_ref[jnp.maximum(g - 1, 0)], -1)
    ne = ne_ref[g]
    par = par_ref[g]
    npar = 1 - par
    nv = nv_ref[0]
    new_expert = e != prev

    @pl.when(new_expert & (g == 0))
    def _():
        c1 = pltpu.make_async_copy(w1_ref.at[e], w1f.at[0], sem1.at[0])
        c2 = pltpu.make_async_copy(w2_ref.at[e], w2f.at[0], sem2.at[0])
        c1.start()
        c2.start()
        c1.wait()
        c2.wait()

    @pl.when(new_expert & (g > 0) & (g < nv))
    def _():
        pltpu.make_async_copy(w1_ref.at[e], w1f.at[par], sem1.at[par]).wait()
        pltpu.make_async_copy(w2_ref.at[e], w2f.at[par], sem2.at[par]).wait()

    @pl.when(new_expert & (ne != e))
    def _():
        pltpu.make_async_copy(w1_ref.at[ne], w1f.at[npar], sem1.at[npar]).start()
        pltpu.make_async_copy(w2_ref.at[ne], w2f.at[npar], sem2.at[npar]).start()

    @pl.when(new_expert & (g < nv))
    def _():
        w1c[...] = w1f[par].astype(jnp.bfloat16)
        w2c[...] = w2f[par].astype(jnp.bfloat16)

    @pl.when(g < nv)
    def _():
        xb = x_ref[...].astype(jnp.bfloat16)
        h = jnp.dot(xb, w1c[...], preferred_element_type=jnp.float32)
        h = jax.nn.silu(h + b1_ref[0, 0])
        y = jnp.dot(
            h.astype(jnp.bfloat16), w2c[...], preferred_element_type=jnp.float32
        )
        y_ref[...] = y + b2_ref[0, 0]


def _grouped_mlp(e_of_g, next_e, parity, nvalid, x, W1, b1, W2, b2, BM, G):
    R = x.shape[0]
    D_IN, D_H = W1.shape[1], W1.shape[2]
    D_OUT = W2.shape[2]
    grid_spec = pltpu.PrefetchScalarGridSpec(
        num_scalar_prefetch=4,
        grid=(G,),
        in_specs=[
            pl.BlockSpec((BM, D_IN), lambda g, e, ne, par, nv: (g, 0)),
            pl.BlockSpec(memory_space=pl.ANY),
            pl.BlockSpec((1, 1, D_H), lambda g, e, ne, par, nv: (e[g], 0, 0)),
            pl.BlockSpec(memory_space=pl.ANY),
            pl.BlockSpec((1, 1, D_OUT), lambda g, e, ne, par, nv: (e[g], 0, 0)),
        ],
        out_specs=pl.BlockSpec((BM, D_OUT), lambda g, e, ne, par, nv: (g, 0)),
        scratch_shapes=[
            pltpu.VMEM((2, D_IN, D_H), jnp.float32),
            pltpu.VMEM((2, D_H, D_OUT), jnp.float32),
            pltpu.VMEM((D_IN, D_H), jnp.bfloat16),
            pltpu.VMEM((D_H, D_OUT), jnp.bfloat16),
            pltpu.SemaphoreType.DMA((2,)),
            pltpu.SemaphoreType.DMA((2,)),
        ],
    )
    return pl.pallas_call(
        _mlp_body,
        grid_spec=grid_spec,
        out_shape=jax.ShapeDtypeStruct((R, D_OUT), jnp.float32),
    )(e_of_g, next_e, parity, nvalid, x, W1, b1[:, None, :], W2, b2[:, None, :])


def kernel(density, species, W1, b1, W2, b2):
    N, D_IN = density.shape
    E = W1.shape[0]
    BM = 128
    G = N // BM + E
    R = G * BM

    e_of_g, next_e, parity, nvalid, pos = _routing(species, N, E, BM, G)
    b_per_w = N // _NW
    C = _chunk_rows(b_per_w, density.shape[1] * 4)
    pos3 = pos.reshape(_NW, b_per_w // C, C)
    gathered = _sc_disperse(density, pos3, R)
    y = _grouped_mlp(
        e_of_g, next_e, parity, nvalid, gathered, W1, b1, W2, b2, BM, G
    )
    return _sc_gather(y, pos3)


# speedup vs baseline: 1.1958x; 1.1958x over previous
import functools

import jax
import jax.numpy as jnp
from jax import lax
from jax.experimental import pallas as pl
from jax.experimental.pallas import tpu as pltpu
from jax.experimental.pallas import tpu_sc as plsc

_SC_CORES = 2
_SC_SUBCORES = 16
_NW = _SC_CORES * _SC_SUBCORES


def _routing(species, N, E, BM, G):
    R = G * BM
    one_hot = (species[:, None] == jnp.arange(E, dtype=species.dtype)).astype(
        jnp.int32
    )
    counts = one_hot.sum(axis=0)
    excl = jnp.cumsum(one_hot, axis=0) - one_hot
    nblk = (counts + BM - 1) // BM
    blk_end = jnp.cumsum(nblk)
    padded_start = (blk_end - nblk) * BM
    pos = ((excl + padded_start[None, :]) * one_hot).sum(axis=1).astype(jnp.int32)
    g = jnp.arange(G, dtype=jnp.int32)
    e_of_g = jnp.minimum(
        (g[:, None] >= blk_end[None, :]).sum(axis=1), E - 1
    ).astype(jnp.int32)
    nvalid = blk_end[E - 1]
    next_e = jnp.where(
        blk_end[e_of_g] >= nvalid,
        e_of_g,
        e_of_g[jnp.clip(blk_end[e_of_g], 0, G - 1)],
    ).astype(jnp.int32)
    trans = jnp.concatenate(
        [jnp.zeros((1,), jnp.int32), (e_of_g[1:] != e_of_g[:-1]).astype(jnp.int32)]
    )
    parity = (jnp.cumsum(trans) % 2).astype(jnp.int32)
    nvalid_arr = jnp.full((G,), nvalid, dtype=jnp.int32)
    return e_of_g, next_e, parity, nvalid_arr, pos


def _chunk_rows(b_per_w, row_bytes, budget=230 * 1024):
    cmax = max(8, min(b_per_w, budget // row_bytes))
    for c in range(cmax - cmax % 8, 0, -8):
        if b_per_w % c == 0:
            return c
    return 8


def _sc_disperse(table, idx3, R):
    NT, D = table.shape
    _, NCH, C = idx3.shape
    b_per_w = NT // _NW
    mesh = plsc.VectorSubcoreMesh(core_axis_name="c", subcore_axis_name="s")

    @functools.partial(
        pl.kernel,
        mesh=mesh,
        out_type=jax.ShapeDtypeStruct((R, D), table.dtype),
        scratch_types=[
            pltpu.VMEM((NCH, C), jnp.int32),
            pltpu.VMEM((C, D), table.dtype),
            pltpu.VMEM((C, D), table.dtype),
            pltpu.SemaphoreType.DMA,
            pltpu.SemaphoreType.DMA,
        ],
    )
    def disperse_k(table_hbm, idx_hbm, out_hbm, idx_v, r0, r1, rsem, wsem):
        wid = lax.axis_index("s") * _SC_CORES + lax.axis_index("c")
        base = wid * b_per_w
        rows_v = (r0, r1)
        reads = [None] * NCH
        writes = [None] * NCH
        pltpu.sync_copy(idx_hbm.at[wid], idx_v)
        reads[0] = pltpu.async_copy(table_hbm.at[pl.ds(base, C)], rows_v[0], rsem)
        for j in range(NCH):
            if j + 1 < NCH:
                if j >= 1:
                    writes[j - 1].wait()
                reads[j + 1] = pltpu.async_copy(
                    table_hbm.at[pl.ds(base + (j + 1) * C, C)],
                    rows_v[(j + 1) % 2],
                    rsem,
                )
            reads[j].wait()
            writes[j] = pltpu.async_copy(
                rows_v[j % 2], out_hbm.at[idx_v.at[j]], wsem
            )
        for j in range(max(0, NCH - 2), NCH):
            writes[j].wait()

    return disperse_k(table, idx3)


def _sc_gather(table, idx3):
    _, D = table.shape
    NW, NCH, C = idx3.shape
    b_per_w = NCH * C
    NT = NW * b_per_w
    mesh = plsc.VectorSubcoreMesh(core_axis_name="c", subcore_axis_name="s")

    @functools.partial(
        pl.kernel,
        mesh=mesh,
        out_type=jax.ShapeDtypeStruct((NT, D), table.dtype),
        scratch_types=[
            pltpu.VMEM((NCH, C), jnp.int32),
            pltpu.VMEM((C, D), table.dtype),
            pltpu.VMEM((C, D), table.dtype),
            pltpu.SemaphoreType.DMA,
            pltpu.SemaphoreType.DMA,
        ],
    )
    def gather_k(table_hbm, idx_hbm, out_hbm, idx_v, r0, r1, gsem, wsem):
        wid = lax.axis_index("s") * _SC_CORES + lax.axis_index("c")
        base = wid * b_per_w
        rows_v = (r0, r1)
        gathers = [None] * NCH
        writes = [None] * NCH
        pltpu.sync_copy(idx_hbm.at[wid], idx_v)
        gathers[0] = pltpu.async_copy(table_hbm.at[idx_v.at[0]], rows_v[0], gsem)
        for j in range(NCH):
            if j + 1 < NCH:
                if j >= 1:
                    writes[j - 1].wait()
                gathers[j + 1] = pltpu.async_copy(
                    table_hbm.at[idx_v.at[j + 1]], rows_v[(j + 1) % 2], gsem
                )
            gathers[j].wait()
            writes[j] = pltpu.async_copy(
                rows_v[j % 2], out_hbm.at[pl.ds(base + j * C, C)], wsem
            )
        for j in range(max(0, NCH - 2), NCH):
            writes[j].wait()

    return gather_k(table, idx3)


def _mlp_body(
    e_ref, ne_ref, par_ref, nv_ref, x_ref, w1_ref, b1_ref, w2_ref, b2_ref, y_ref,
    w1f, w2f, w1c, w2c, sem1, sem2,
):
    g = pl.program_id(0)
    e = e_ref[g]
    prev = jnp.where(g > 0, e_ref[jnp.maximum(g - 1, 0)], -1)
    ne = ne_ref[g]
    par = par_ref[g]
    npar = 1 - par
    nv = nv_ref[0]
    new_expert = e != prev

    @pl.when(new_expert & (g == 0))
    def _():
        c1 = pltpu.make_async_copy(w1_ref.at[e], w1f.at[0], sem1.at[0])
        c2 = pltpu.make_async_copy(w2_ref.at[e], w2f.at[0], sem2.at[0])
        c1.start()
        c2.start()
        c1.wait()
        c2.wait()

    @pl.when(new_expert & (g > 0) & (g < nv))
    def _():
        pltpu.make_async_copy(w1_ref.at[e], w1f.at[par], sem1.at[par]).wait()
        pltpu.make_async_copy(w2_ref.at[e], w2f.at[par], sem2.at[par]).wait()

    @pl.when(new_expert & (ne != e))
    def _():
        pltpu.make_async_copy(w1_ref.at[ne], w1f.at[npar], sem1.at[npar]).start()
        pltpu.make_async_copy(w2_ref.at[ne], w2f.at[npar], sem2.at[npar]).start()

    @pl.when(new_expert & (g < nv))
    def _():
        w1c[...] = w1f[par].astype(jnp.bfloat16)
        w2c[...] = w2f[par].astype(jnp.bfloat16)

    @pl.when(g < nv)
    def _():
        xb = x_ref[...].astype(jnp.bfloat16)
        h = jnp.dot(xb, w1c[...], preferred_element_type=jnp.float32)
        h = jax.nn.silu(h + b1_ref[0, 0])
        y = jnp.dot(
            h.astype(jnp.bfloat16), w2c[...], preferred_element_type=jnp.float32
        )
        y_ref[...] = y + b2_ref[0, 0]


def _grouped_mlp(e_of_g, next_e, parity, nvalid, x, W1, b1, W2, b2, BM, G):
    R = x.shape[0]
    D_IN, D_H = W1.shape[1], W1.shape[2]
    D_OUT = W2.shape[2]
    grid_spec = pltpu.PrefetchScalarGridSpec(
        num_scalar_prefetch=4,
        grid=(G,),
        in_specs=[
            pl.BlockSpec((BM, D_IN), lambda g, e, ne, par, nv: (g, 0)),
            pl.BlockSpec(memory_space=pl.ANY),
            pl.BlockSpec((1, 1, D_H), lambda g, e, ne, par, nv: (e[g], 0, 0)),
            pl.BlockSpec(memory_space=pl.ANY),
            pl.BlockSpec((1, 1, D_OUT), lambda g, e, ne, par, nv: (e[g], 0, 0)),
        ],
        out_specs=pl.BlockSpec((BM, D_OUT), lambda g, e, ne, par, nv: (g, 0)),
        scratch_shapes=[
            pltpu.VMEM((2, D_IN, D_H), jnp.float32),
            pltpu.VMEM((2, D_H, D_OUT), jnp.float32),
            pltpu.VMEM((D_IN, D_H), jnp.bfloat16),
            pltpu.VMEM((D_H, D_OUT), jnp.bfloat16),
            pltpu.SemaphoreType.DMA((2,)),
            pltpu.SemaphoreType.DMA((2,)),
        ],
    )
    return pl.pallas_call(
        _mlp_body,
        grid_spec=grid_spec,
        out_shape=jax.ShapeDtypeStruct((R, D_OUT), jnp.float32),
    )(e_of_g, next_e, parity, nvalid, x, W1, b1[:, None, :], W2, b2[:, None, :])


def kernel(density, species, W1, b1, W2, b2):
    N, D_IN = density.shape
    E = W1.shape[0]
    BM = 256
    G = N // BM + E
    R = G * BM

    e_of_g, next_e, parity, nvalid, pos = _routing(species, N, E, BM, G)
    b_per_w = N // _NW
    C = _chunk_rows(b_per_w, density.shape[1] * 4)
    pos3 = pos.reshape(_NW, b_per_w // C, C)
    gathered = _sc_disperse(density, pos3, R)
    y = _grouped_mlp(
        e_of_g, next_e, parity, nvalid, gathered, W1, b1, W2, b2, BM, G
    )
    return _sc_gather(y, pos3)


# speedup vs baseline: 1.1985x; 1.0023x over previous
import functools

import jax
import jax.numpy as jnp
from jax import lax
from jax.experimental import pallas as pl
from jax.experimental.pallas import tpu as pltpu
from jax.experimental.pallas import tpu_sc as plsc

_SC_CORES = 2
_SC_SUBCORES = 16
_NW = _SC_CORES * _SC_SUBCORES


def _routing(species, N, E, BM, G):
    R = G * BM
    one_hot = (species[:, None] == jnp.arange(E, dtype=species.dtype)).astype(
        jnp.int32
    )
    counts = one_hot.sum(axis=0)
    excl = jnp.cumsum(one_hot, axis=0) - one_hot
    nblk = (counts + BM - 1) // BM
    blk_end = jnp.cumsum(nblk)
    padded_start = (blk_end - nblk) * BM
    pos = ((excl + padded_start[None, :]) * one_hot).sum(axis=1).astype(jnp.int32)
    g = jnp.arange(G, dtype=jnp.int32)
    e_of_g = jnp.minimum(
        (g[:, None] >= blk_end[None, :]).sum(axis=1), E - 1
    ).astype(jnp.int32)
    nvalid = blk_end[E - 1]
    next_e = jnp.where(
        blk_end[e_of_g] >= nvalid,
        e_of_g,
        e_of_g[jnp.clip(blk_end[e_of_g], 0, G - 1)],
    ).astype(jnp.int32)
    trans = jnp.concatenate(
        [jnp.zeros((1,), jnp.int32), (e_of_g[1:] != e_of_g[:-1]).astype(jnp.int32)]
    )
    parity = (jnp.cumsum(trans) % 2).astype(jnp.int32)
    nvalid_arr = jnp.full((G,), nvalid, dtype=jnp.int32)
    return e_of_g, next_e, parity, nvalid_arr, pos


def _chunk_rows(b_per_w, row_bytes, budget=230 * 1024):
    cmax = max(8, min(b_per_w, budget // row_bytes))
    for c in range(cmax - cmax % 8, 0, -8):
        if b_per_w % c == 0:
            return c
    return 8


def _sc_disperse(table, idx3, R):
    NT, D = table.shape
    _, NCH, C = idx3.shape
    b_per_w = NT // _NW
    mesh = plsc.VectorSubcoreMesh(core_axis_name="c", subcore_axis_name="s")

    @functools.partial(
        pl.kernel,
        mesh=mesh,
        out_type=jax.ShapeDtypeStruct((R, D), table.dtype),
        scratch_types=[
            pltpu.VMEM((NCH, C), jnp.int32),
            pltpu.VMEM((C, D), table.dtype),
            pltpu.VMEM((C, D), table.dtype),
            pltpu.SemaphoreType.DMA,
            pltpu.SemaphoreType.DMA,
        ],
    )
    def disperse_k(table_hbm, idx_hbm, out_hbm, idx_v, r0, r1, rsem, wsem):
        wid = lax.axis_index("s") * _SC_CORES + lax.axis_index("c")
        base = wid * b_per_w
        rows_v = (r0, r1)
        reads = [None] * NCH
        writes = [None] * NCH
        pltpu.sync_copy(idx_hbm.at[wid], idx_v)
        reads[0] = pltpu.async_copy(table_hbm.at[pl.ds(base, C)], rows_v[0], rsem)
        for j in range(NCH):
            if j + 1 < NCH:
                if j >= 1:
                    writes[j - 1].wait()
                reads[j + 1] = pltpu.async_copy(
                    table_hbm.at[pl.ds(base + (j + 1) * C, C)],
                    rows_v[(j + 1) % 2],
                    rsem,
                )
            reads[j].wait()
            writes[j] = pltpu.async_copy(
                rows_v[j % 2], out_hbm.at[idx_v.at[j]], wsem
            )
        for j in range(max(0, NCH - 2), NCH):
            writes[j].wait()

    return disperse_k(table, idx3)


def _sc_gather(table, idx3):
    _, D = table.shape
    NW, NCH, C = idx3.shape
    b_per_w = NCH * C
    NT = NW * b_per_w
    mesh = plsc.VectorSubcoreMesh(core_axis_name="c", subcore_axis_name="s")

    @functools.partial(
        pl.kernel,
        mesh=mesh,
        out_type=jax.ShapeDtypeStruct((NT, D), table.dtype),
        scratch_types=[
            pltpu.VMEM((NCH, C), jnp.int32),
            pltpu.VMEM((C, D), table.dtype),
            pltpu.VMEM((C, D), table.dtype),
            pltpu.SemaphoreType.DMA,
            pltpu.SemaphoreType.DMA,
        ],
    )
    def gather_k(table_hbm, idx_hbm, out_hbm, idx_v, r0, r1, gsem, wsem):
        wid = lax.axis_index("s") * _SC_CORES + lax.axis_index("c")
        base = wid * b_per_w
        rows_v = (r0, r1)
        gathers = [None] * NCH
        writes = [None] * NCH
        pltpu.sync_copy(idx_hbm.at[wid], idx_v)
        gathers[0] = pltpu.async_copy(table_hbm.at[idx_v.at[0]], rows_v[0], gsem)
        for j in range(NCH):
            if j + 1 < NCH:
                if j >= 1:
                    writes[j - 1].wait()
                gathers[j + 1] = pltpu.async_copy(
                    table_hbm.at[idx_v.at[j + 1]], rows_v[(j + 1) % 2], gsem
                )
            gathers[j].wait()
            writes[j] = pltpu.async_copy(
                rows_v[j % 2], out_hbm.at[pl.ds(base + j * C, C)], wsem
            )
        for j in range(max(0, NCH - 2), NCH):
            writes[j].wait()

    return gather_k(table, idx3)


def _mlp_body(
    e_ref, ne_ref, par_ref, nv_ref, x_ref, w1_ref, b1_ref, w2_ref, b2_ref, y_ref,
    w1f, w2f, w1c, w2c, sem1, sem2,
):
    g = pl.program_id(0)
    e = e_ref[g]
    prev = jnp.where(g > 0, e_ref[jnp.maximum(g - 1, 0)], -1)
    ne = ne_ref[g]
    par = par_ref[g]
    npar = 1 - par
    nv = nv_ref[0]
    new_expert = e != prev

    @pl.when(new_expert & (g == 0))
    def _():
        c1 = pltpu.make_async_copy(w1_ref.at[e], w1f.at[0], sem1.at[0])
        c2 = pltpu.make_async_copy(w2_ref.at[e], w2f.at[0], sem2.at[0])
        c1.start()
        c2.start()
        c1.wait()
        c2.wait()

    @pl.when(new_expert & (g > 0) & (g < nv))
    def _():
        pltpu.make_async_copy(w1_ref.at[e], w1f.at[par], sem1.at[par]).wait()
        pltpu.make_async_copy(w2_ref.at[e], w2f.at[par], sem2.at[par]).wait()

    @pl.when(new_expert & (ne != e))
    def _():
        pltpu.make_async_copy(w1_ref.at[ne], w1f.at[npar], sem1.at[npar]).start()
        pltpu.make_async_copy(w2_ref.at[ne], w2f.at[npar], sem2.at[npar]).start()

    @pl.when(new_expert & (g < nv))
    def _():
        w1c[...] = w1f[par].astype(jnp.bfloat16)
        w2c[...] = w2f[par].astype(jnp.bfloat16)

    @pl.when(g < nv)
    def _():
        xb = x_ref[...].astype(jnp.bfloat16)
        h = jnp.dot(xb, w1c[...], preferred_element_type=jnp.float32)
        h = jax.nn.silu(h + b1_ref[e, 0])
        y = jnp.dot(
            h.astype(jnp.bfloat16), w2c[...], preferred_element_type=jnp.float32
        )
        y_ref[...] = y + b2_ref[e, 0]


def _grouped_mlp(e_of_g, next_e, parity, nvalid, x, W1, b1, W2, b2, BM, G):
    R = x.shape[0]
    E, D_IN, D_H = W1.shape
    D_OUT = W2.shape[2]
    grid_spec = pltpu.PrefetchScalarGridSpec(
        num_scalar_prefetch=4,
        grid=(G,),
        in_specs=[
            pl.BlockSpec((BM, D_IN), lambda g, e, ne, par, nv: (g, 0)),
            pl.BlockSpec(memory_space=pl.ANY),
            pl.BlockSpec((E, 1, D_H), lambda g, e, ne, par, nv: (0, 0, 0)),
            pl.BlockSpec(memory_space=pl.ANY),
            pl.BlockSpec((E, 1, D_OUT), lambda g, e, ne, par, nv: (0, 0, 0)),
        ],
        out_specs=pl.BlockSpec((BM, D_OUT), lambda g, e, ne, par, nv: (g, 0)),
        scratch_shapes=[
            pltpu.VMEM((2, D_IN, D_H), jnp.float32),
            pltpu.VMEM((2, D_H, D_OUT), jnp.float32),
            pltpu.VMEM((D_IN, D_H), jnp.bfloat16),
            pltpu.VMEM((D_H, D_OUT), jnp.bfloat16),
            pltpu.SemaphoreType.DMA((2,)),
            pltpu.SemaphoreType.DMA((2,)),
        ],
    )
    return pl.pallas_call(
        _mlp_body,
        grid_spec=grid_spec,
        out_shape=jax.ShapeDtypeStruct((R, D_OUT), jnp.float32),
    )(e_of_g, next_e, parity, nvalid, x, W1, b1[:, None, :], W2, b2[:, None, :])


def kernel(density, species, W1, b1, W2, b2):
    N, D_IN = density.shape
    E = W1.shape[0]
    BM = 256
    G = N // BM + E
    R = G * BM

    e_of_g, next_e, parity, nvalid, pos = _routing(species, N, E, BM, G)
    b_per_w = N // _NW
    C = _chunk_rows(b_per_w, density.shape[1] * 4)
    pos3 = pos.reshape(_NW, b_per_w // C, C)
    gathered = _sc_disperse(density, pos3, R)
    y = _grouped_mlp(
        e_of_g, next_e, parity, nvalid, gathered, W1, b1, W2, b2, BM, G
    )
    return _sc_gather(y, pos3)
